# baseline (speedup 1.0000x reference)
import functools

import jax
import jax.numpy as jnp
from jax import lax
from jax.experimental import pallas as pl
from jax.experimental.pallas import tpu as pltpu
from jax.experimental.pallas import tpu_sc as plsc

N_EXP = 16
D = 1024
B = 8192
T = 256
NT = B // T + N_EXP
P = NT * T
NEG = -3.0e38

NC = 2
NS = 16
NW = NC * NS
RPW = B // NW
CH = 32
NCH = RPW // CH


def _router_body(act_ref, r_ref, rb_ref,
                 maxp_ref, idx_ref, wsum_ref, pos_ref, te_ref, tv_ref,
                 prop_ref, idx_sc, rank_sc, cnt_sc):
    i = pl.program_id(0)
    x = act_ref[...] - rb_ref[...]
    logits = jnp.dot(x, r_ref[...], preferred_element_type=jnp.float32)
    m = jnp.max(logits, axis=-1, keepdims=True)
    p = jnp.exp(logits - m)
    probs = p / jnp.sum(p, axis=-1, keepdims=True)
    maxp_ref[...] = jnp.max(probs, axis=-1, keepdims=True)
    lane = lax.broadcasted_iota(jnp.int32, (T, N_EXP), 1)
    eidx = jnp.min(jnp.where(logits == m, lane, N_EXP), axis=-1,
                   keepdims=True)
    idx_ref[...] = eidx
    idx_sc[pl.ds(i * T, T), :] = eidx
    onehot = (lane == eidx).astype(jnp.float32)

    @pl.when(i == 0)
    def _():
        cnt_sc[...] = jnp.zeros((1, N_EXP), jnp.float32)
        wsum_ref[...] = jnp.zeros((1, N_EXP), jnp.float32)

    tri = (lax.broadcasted_iota(jnp.int32, (T, T), 0)
           >= lax.broadcasted_iota(jnp.int32, (T, T), 1)).astype(jnp.float32)
    csum = jnp.dot(tri, onehot, preferred_element_type=jnp.float32) \
        + cnt_sc[...]
    rank = jnp.sum(onehot * csum, axis=-1, keepdims=True) - 1.0
    rank_sc[pl.ds(i * T, T), :] = rank.astype(jnp.int32)
    cnt_sc[...] = cnt_sc[...] + jnp.sum(onehot, axis=0, keepdims=True)
    wsum_ref[...] = wsum_ref[...] + jnp.sum(probs, axis=0, keepdims=True)

    @pl.when(i == B // T - 1)
    def _():
        wsum_ref[...] = wsum_ref[...] / float(B)
        prop_ref[...] = cnt_sc[...] / float(B)
        t_iota = lax.broadcasted_iota(jnp.int32, (1, NT), 1) \
            .astype(jnp.float32)
        te_f = jnp.zeros((1, NT), jnp.float32)
        start = 0.0
        starts = []
        bases = []
        cs = []
        for e in range(N_EXP):
            c = cnt_sc[0, e]
            cs.append(c)
            starts.append(start)
            bases.append(start / float(T))
            aligned = jnp.ceil(c / float(T)) * float(T)
            start = start + aligned
            te_f = te_f + (t_iota >= start / float(T)).astype(jnp.float32)
        te_f = jnp.minimum(te_f, float(N_EXP - 1))
        tv_f = jnp.zeros((1, NT), jnp.float32)
        for e in range(N_EXP):
            rem = cs[e] - (t_iota - bases[e]) * float(T)
            rem = jnp.clip(rem, 0.0, float(T))
            tv_f = tv_f + jnp.where(te_f == float(e), rem, 0.0)
        te_ref[...] = te_f.astype(jnp.int32)
        tv_ref[...] = tv_f.astype(jnp.int32)

        allidx = idx_sc[...]
        blane = lax.broadcasted_iota(jnp.int32, (B, N_EXP), 1)
        bonehot = (allidx == blane).astype(jnp.float32)
        erow = lax.broadcasted_iota(jnp.int32, (N_EXP, 1), 0)
        start_col = jnp.zeros((N_EXP, 1), jnp.float32)
        for e in range(N_EXP):
            start_col = start_col + jnp.where(erow == e, starts[e], 0.0)
        start_sel = jnp.dot(bonehot, start_col,
                            preferred_element_type=jnp.float32)
        pos_ref[...] = rank_sc[...] + start_sel.astype(jnp.int32)


_router_call = pl.pallas_call(
    _router_body,
    grid=(B // T,),
    in_specs=[
        pl.BlockSpec((T, D), lambda i: (i, 0)),
        pl.BlockSpec((D, N_EXP), lambda i: (0, 0)),
        pl.BlockSpec((1, D), lambda i: (0, 0)),
    ],
    out_specs=[
        pl.BlockSpec((T, 1), lambda i: (i, 0)),
        pl.BlockSpec((T, 1), lambda i: (i, 0)),
        pl.BlockSpec((1, N_EXP), lambda i: (0, 0)),
        pl.BlockSpec((B, 1), lambda i: (0, 0)),
        pl.BlockSpec((1, NT), lambda i: (0, 0)),
        pl.BlockSpec((1, NT), lambda i: (0, 0)),
        pl.BlockSpec((1, N_EXP), lambda i: (0, 0)),
    ],
    out_shape=[
        jax.ShapeDtypeStruct((B, 1), jnp.float32),
        jax.ShapeDtypeStruct((B, 1), jnp.int32),
        jax.ShapeDtypeStruct((1, N_EXP), jnp.float32),
        jax.ShapeDtypeStruct((B, 1), jnp.int32),
        jax.ShapeDtypeStruct((1, NT), jnp.int32),
        jax.ShapeDtypeStruct((1, NT), jnp.int32),
        jax.ShapeDtypeStruct((1, N_EXP), jnp.float32),
    ],
    scratch_shapes=[
        pltpu.VMEM((B, 1), jnp.int32),
        pltpu.VMEM((B, 1), jnp.int32),
        pltpu.VMEM((1, N_EXP), jnp.float32),
    ],
)


_sc_mesh = plsc.VectorSubcoreMesh(core_axis_name="c", subcore_axis_name="s")


@functools.partial(
    pl.kernel,
    mesh=_sc_mesh,
    out_type=jax.ShapeDtypeStruct((P, D), jnp.float32),
    scratch_types=[
        pltpu.VMEM((NCH, CH), jnp.int32),
        pltpu.VMEM((CH, D), jnp.float32),
        pltpu.VMEM((CH, D), jnp.float32),
        pltpu.SemaphoreType.DMA,
        pltpu.SemaphoreType.DMA,
        pltpu.SemaphoreType.DMA,
        pltpu.SemaphoreType.DMA,
    ],
)
def _sc_scatter(act_hbm, pos_hbm, out_hbm, idx_v, b0, b1, l0, l1, s0, s1):
    wid = lax.axis_index("s") * NC + lax.axis_index("c")
    base = wid * RPW
    pltpu.sync_copy(pos_hbm.at[pl.ds(wid * NCH, NCH)], idx_v)
    bufs, lsem, ssem = (b0, b1), (l0, l1), (s0, s1)
    loads = [None] * NCH
    scats = [None] * NCH
    loads[0] = pltpu.async_copy(act_hbm.at[pl.ds(base, CH)], bufs[0],
                                lsem[0])
    for c in range(NCH):
        k = c % 2
        if c + 1 < NCH:
            nk = (c + 1) % 2
            if c >= 1:
                scats[c - 1].wait()
            loads[c + 1] = pltpu.async_copy(
                act_hbm.at[pl.ds(base + (c + 1) * CH, CH)], bufs[nk],
                lsem[nk])
        loads[c].wait()
        scats[c] = pltpu.async_copy(bufs[k], out_hbm.at[idx_v.at[c]],
                                    ssem[k])
    scats[NCH - 2].wait()
    scats[NCH - 1].wait()


def _gmm_body(te_ref, tv_ref, xs_ref, enc_ref, encb_ref, pb_ref,
              lat_ref, rec_ref, wa_ref):
    t = pl.program_id(0)
    e = te_ref[t]
    tv = tv_ref[t]

    @pl.when(t == 0)
    def _():
        wa_ref[...] = jnp.full((N_EXP, D), NEG, jnp.float32)

    @pl.when(tv > 0)
    def _():
        x = xs_ref[...] - pb_ref[...]
        lat = jnp.maximum(
            jnp.dot(x, enc_ref[0], preferred_element_type=jnp.float32), 0.0)
        rec = lax.dot_general(
            lat.astype(jnp.bfloat16), encb_ref[0], (((1,), (1,)), ((), ())),
            preferred_element_type=jnp.float32)
        lat_ref[...] = lat
        rec_ref[...] = rec
        rows = lax.broadcasted_iota(jnp.int32, (T, 1), 0)
        masked = jnp.where(rows < tv, lat, NEG)
        m = jnp.max(masked, axis=0, keepdims=True)
        cur = wa_ref[pl.ds(e, 1), :]
        wa_ref[pl.ds(e, 1), :] = jnp.maximum(cur, m)

    @pl.when(t == NT - 1)
    def _():
        wa_ref[...] = jnp.where(wa_ref[...] > 0.001, 1.0, 0.0)


_gmm_call = pl.pallas_call(
    _gmm_body,
    grid_spec=pltpu.PrefetchScalarGridSpec(
        num_scalar_prefetch=2,
        grid=(NT,),
        in_specs=[
            pl.BlockSpec((T, D), lambda t, te, tv: (t, 0)),
            pl.BlockSpec((1, D, D), lambda t, te, tv: (te[t], 0, 0)),
            pl.BlockSpec((1, D, D), lambda t, te, tv: (te[t], 0, 0)),
            pl.BlockSpec((1, D), lambda t, te, tv: (0, 0)),
        ],
        out_specs=[
            pl.BlockSpec((T, D), lambda t, te, tv: (t, 0)),
            pl.BlockSpec((T, D), lambda t, te, tv: (t, 0)),
            pl.BlockSpec((N_EXP, D), lambda t, te, tv: (0, 0)),
        ],
    ),
    out_shape=[
        jax.ShapeDtypeStruct((P, D), jnp.float32),
        jax.ShapeDtypeStruct((P, D), jnp.float32),
        jax.ShapeDtypeStruct((N_EXP, D), jnp.float32),
    ],
)


@functools.partial(
    pl.kernel,
    mesh=_sc_mesh,
    out_type=jax.ShapeDtypeStruct((B, D), jnp.float32),
    scratch_types=[
        pltpu.VMEM((NCH, CH), jnp.int32),
        pltpu.VMEM((CH, D), jnp.float32),
        pltpu.VMEM((CH, D), jnp.float32),
        pltpu.SemaphoreType.DMA,
        pltpu.SemaphoreType.DMA,
        pltpu.SemaphoreType.DMA,
        pltpu.SemaphoreType.DMA,
    ],
)
def _sc_gather(src_hbm, pos_hbm, out_hbm, idx_v, b0, b1, g0, g1, o0, o1):
    wid = lax.axis_index("s") * NC + lax.axis_index("c")
    base = wid * RPW
    pltpu.sync_copy(pos_hbm.at[pl.ds(wid * NCH, NCH)], idx_v)
    bufs, gsem, osem = (b0, b1), (g0, g1), (o0, o1)
    gats = [None] * NCH
    outs = [None] * NCH
    gats[0] = pltpu.async_copy(src_hbm.at[idx_v.at[0]], bufs[0], gsem[0])
    for c in range(NCH):
        k = c % 2
        if c + 1 < NCH:
            nk = (c + 1) % 2
            if c >= 1:
                outs[c - 1].wait()
            gats[c + 1] = pltpu.async_copy(src_hbm.at[idx_v.at[c + 1]],
                                           bufs[nk], gsem[nk])
        gats[c].wait()
        outs[c] = pltpu.async_copy(bufs[k],
                                   out_hbm.at[pl.ds(base + c * CH, CH)],
                                   osem[k])
    outs[NCH - 2].wait()
    outs[NCH - 1].wait()


def _combine_body(rec_ref, maxp_ref, tok_ref, pb_ref, out_ref):
    out_ref[...] = (maxp_ref[...] * rec_ref[...] + tok_ref[...]
                    + pb_ref[...])


_combine_call = pl.pallas_call(
    _combine_body,
    grid=(B // T,),
    in_specs=[
        pl.BlockSpec((T, D), lambda i: (i, 0)),
        pl.BlockSpec((T, 1), lambda i: (i, 0)),
        pl.BlockSpec((T, D), lambda i: (i, 0)),
        pl.BlockSpec((1, D), lambda i: (0, 0)),
    ],
    out_specs=pl.BlockSpec((T, D), lambda i: (i, 0)),
    out_shape=jax.ShapeDtypeStruct((B, D), jnp.float32),
)


def kernel(activations, token_act, pre_b, enc, dec, router_b, router):
    pb2 = pre_b.reshape(1, D)
    maxp, eidx, wmean, pos, te, tv, prop = _router_call(
        activations, router, router_b.reshape(1, D))
    pos2 = pos.reshape(B // CH, CH)
    sorted_a = _sc_scatter(activations, pos2)
    lat_s, rec_s, wa = _gmm_call(
        te.reshape(NT), tv.reshape(NT), sorted_a, enc,
        enc.astype(jnp.bfloat16), pb2)
    rec_g = _sc_gather(rec_s, pos2)
    reconstruction = _combine_call(rec_g, maxp, token_act, pb2)
    full_latent = _sc_gather(lat_s, pos2)
    return (reconstruction, full_latent, wa.astype(bool),
            eidx.reshape(B), prop.reshape(N_EXP), wmean.reshape(N_EXP))


# speedup vs baseline: 1.1450x; 1.1450x over previous
import functools

import jax
import jax.numpy as jnp
from jax import lax
from jax.experimental import pallas as pl
from jax.experimental.pallas import tpu as pltpu
from jax.experimental.pallas import tpu_sc as plsc

N_EXP = 16
D = 1024
B = 8192
T = 256
NT = B // T + N_EXP
P = NT * T
NEG = -3.0e38

NC = 2
NS = 16
NW = NC * NS
RPW = B // NW
CH = 32
NCH = RPW // CH


def _router_body(act_ref, r_ref, rb_ref,
                 maxp_ref, idx_ref, wsum_ref, pos_ref, te_ref, tv_ref,
                 prop_ref, idx_sc, rank_sc, cnt_sc):
    i = pl.program_id(0)
    x = act_ref[...] - rb_ref[...]
    logits = jnp.dot(x, r_ref[...], preferred_element_type=jnp.float32)
    m = jnp.max(logits, axis=-1, keepdims=True)
    p = jnp.exp(logits - m)
    probs = p / jnp.sum(p, axis=-1, keepdims=True)
    maxp_ref[...] = jnp.max(probs, axis=-1, keepdims=True)
    lane = lax.broadcasted_iota(jnp.int32, (T, N_EXP), 1)
    eidx = jnp.min(jnp.where(logits == m, lane, N_EXP), axis=-1,
                   keepdims=True)
    idx_ref[...] = eidx
    idx_sc[pl.ds(i * T, T), :] = eidx
    onehot = (lane == eidx).astype(jnp.float32)

    @pl.when(i == 0)
    def _():
        cnt_sc[...] = jnp.zeros((1, N_EXP), jnp.float32)
        wsum_ref[...] = jnp.zeros((1, N_EXP), jnp.float32)

    tri = (lax.broadcasted_iota(jnp.int32, (T, T), 0)
           >= lax.broadcasted_iota(jnp.int32, (T, T), 1)).astype(jnp.float32)
    csum = jnp.dot(tri, onehot, preferred_element_type=jnp.float32) \
        + cnt_sc[...]
    rank = jnp.sum(onehot * csum, axis=-1, keepdims=True) - 1.0
    rank_sc[pl.ds(i * T, T), :] = rank.astype(jnp.int32)
    cnt_sc[...] = cnt_sc[...] + jnp.sum(onehot, axis=0, keepdims=True)
    wsum_ref[...] = wsum_ref[...] + jnp.sum(probs, axis=0, keepdims=True)

    @pl.when(i == B // T - 1)
    def _():
        wsum_ref[...] = wsum_ref[...] / float(B)
        prop_ref[...] = cnt_sc[...] / float(B)
        t_iota = lax.broadcasted_iota(jnp.int32, (1, NT), 1) \
            .astype(jnp.float32)
        te_f = jnp.zeros((1, NT), jnp.float32)
        start = 0.0
        starts = []
        bases = []
        cs = []
        for e in range(N_EXP):
            c = cnt_sc[0, e]
            cs.append(c)
            starts.append(start)
            bases.append(start / float(T))
            aligned = jnp.ceil(c / float(T)) * float(T)
            start = start + aligned
            te_f = te_f + (t_iota >= start / float(T)).astype(jnp.float32)
        te_f = jnp.minimum(te_f, float(N_EXP - 1))
        tv_f = jnp.zeros((1, NT), jnp.float32)
        for e in range(N_EXP):
            rem = cs[e] - (t_iota - bases[e]) * float(T)
            rem = jnp.clip(rem, 0.0, float(T))
            tv_f = tv_f + jnp.where(te_f == float(e), rem, 0.0)
        te_ref[...] = te_f.astype(jnp.int32)
        tv_ref[...] = tv_f.astype(jnp.int32)

        allidx = idx_sc[...]
        blane = lax.broadcasted_iota(jnp.int32, (B, N_EXP), 1)
        bonehot = (allidx == blane).astype(jnp.float32)
        erow = lax.broadcasted_iota(jnp.int32, (N_EXP, 1), 0)
        start_col = jnp.zeros((N_EXP, 1), jnp.float32)
        for e in range(N_EXP):
            start_col = start_col + jnp.where(erow == e, starts[e], 0.0)
        start_sel = jnp.dot(bonehot, start_col,
                            preferred_element_type=jnp.float32)
        pos_ref[...] = rank_sc[...] + start_sel.astype(jnp.int32)


_router_call = pl.pallas_call(
    _router_body,
    grid=(B // T,),
    in_specs=[
        pl.BlockSpec((T, D), lambda i: (i, 0)),
        pl.BlockSpec((D, N_EXP), lambda i: (0, 0)),
        pl.BlockSpec((1, D), lambda i: (0, 0)),
    ],
    out_specs=[
        pl.BlockSpec((T, 1), lambda i: (i, 0)),
        pl.BlockSpec((T, 1), lambda i: (i, 0)),
        pl.BlockSpec((1, N_EXP), lambda i: (0, 0)),
        pl.BlockSpec((B, 1), lambda i: (0, 0)),
        pl.BlockSpec((1, NT), lambda i: (0, 0)),
        pl.BlockSpec((1, NT), lambda i: (0, 0)),
        pl.BlockSpec((1, N_EXP), lambda i: (0, 0)),
    ],
    out_shape=[
        jax.ShapeDtypeStruct((B, 1), jnp.float32),
        jax.ShapeDtypeStruct((B, 1), jnp.int32),
        jax.ShapeDtypeStruct((1, N_EXP), jnp.float32),
        jax.ShapeDtypeStruct((B, 1), jnp.int32),
        jax.ShapeDtypeStruct((1, NT), jnp.int32),
        jax.ShapeDtypeStruct((1, NT), jnp.int32),
        jax.ShapeDtypeStruct((1, N_EXP), jnp.float32),
    ],
    scratch_shapes=[
        pltpu.VMEM((B, 1), jnp.int32),
        pltpu.VMEM((B, 1), jnp.int32),
        pltpu.VMEM((1, N_EXP), jnp.float32),
    ],
)


_sc_mesh = plsc.VectorSubcoreMesh(core_axis_name="c", subcore_axis_name="s")


@functools.partial(
    pl.kernel,
    mesh=_sc_mesh,
    out_type=jax.ShapeDtypeStruct((P, D), jnp.float32),
    scratch_types=[
        pltpu.VMEM((NCH, CH), jnp.int32),
        pltpu.VMEM((CH, D), jnp.float32),
        pltpu.VMEM((CH, D), jnp.float32),
        pltpu.SemaphoreType.DMA,
        pltpu.SemaphoreType.DMA,
        pltpu.SemaphoreType.DMA,
        pltpu.SemaphoreType.DMA,
    ],
)
def _sc_scatter(act_hbm, pos_hbm, out_hbm, idx_v, b0, b1, l0, l1, s0, s1):
    wid = lax.axis_index("s") * NC + lax.axis_index("c")
    base = wid * RPW
    pltpu.sync_copy(pos_hbm.at[pl.ds(wid * NCH, NCH)], idx_v)
    bufs, lsem, ssem = (b0, b1), (l0, l1), (s0, s1)
    loads = [None] * NCH
    scats = [None] * NCH
    loads[0] = pltpu.async_copy(act_hbm.at[pl.ds(base, CH)], bufs[0],
                                lsem[0])
    for c in range(NCH):
        k = c % 2
        if c + 1 < NCH:
            nk = (c + 1) % 2
            if c >= 1:
                scats[c - 1].wait()
            loads[c + 1] = pltpu.async_copy(
                act_hbm.at[pl.ds(base + (c + 1) * CH, CH)], bufs[nk],
                lsem[nk])
        loads[c].wait()
        scats[c] = pltpu.async_copy(bufs[k], out_hbm.at[idx_v.at[c]],
                                    ssem[k])
    scats[NCH - 2].wait()
    scats[NCH - 1].wait()


def _gmm_body(te_ref, tv_ref, xs_ref, enc_ref, pb_ref,
              lat_ref, rec_ref, wa_ref):
    t = pl.program_id(0)
    e = te_ref[t]
    tv = tv_ref[t]

    @pl.when(t == 0)
    def _():
        wa_ref[...] = jnp.full((N_EXP, D), NEG, jnp.float32)

    @pl.when(tv > 0)
    def _():
        x = xs_ref[...] - pb_ref[...]
        lat = jnp.maximum(
            jnp.dot(x, enc_ref[0], preferred_element_type=jnp.float32), 0.0)
        rec = lax.dot_general(
            lat, enc_ref[0], (((1,), (1,)), ((), ())),
            preferred_element_type=jnp.float32)
        lat_ref[...] = lat
        rec_ref[...] = rec
        rows = lax.broadcasted_iota(jnp.int32, (T, 1), 0)
        masked = jnp.where(rows < tv, lat, NEG)
        m = jnp.max(masked, axis=0, keepdims=True)
        cur = wa_ref[pl.ds(e, 1), :]
        wa_ref[pl.ds(e, 1), :] = jnp.maximum(cur, m)

    @pl.when(t == NT - 1)
    def _():
        wa_ref[...] = jnp.where(wa_ref[...] > 0.001, 1.0, 0.0)


_gmm_call = pl.pallas_call(
    _gmm_body,
    grid_spec=pltpu.PrefetchScalarGridSpec(
        num_scalar_prefetch=2,
        grid=(NT,),
        in_specs=[
            pl.BlockSpec((T, D), lambda t, te, tv: (t, 0)),
            pl.BlockSpec((1, D, D), lambda t, te, tv: (te[t], 0, 0)),
            pl.BlockSpec((1, D), lambda t, te, tv: (0, 0)),
        ],
        out_specs=[
            pl.BlockSpec((T, D), lambda t, te, tv: (t, 0)),
            pl.BlockSpec((T, D), lambda t, te, tv: (t, 0)),
            pl.BlockSpec((N_EXP, D), lambda t, te, tv: (0, 0)),
        ],
    ),
    out_shape=[
        jax.ShapeDtypeStruct((P, D), jnp.float32),
        jax.ShapeDtypeStruct((P, D), jnp.float32),
        jax.ShapeDtypeStruct((N_EXP, D), jnp.float32),
    ],
)


@functools.partial(
    pl.kernel,
    mesh=_sc_mesh,
    out_type=jax.ShapeDtypeStruct((B, D), jnp.float32),
    scratch_types=[
        pltpu.VMEM((NCH, CH), jnp.int32),
        pltpu.VMEM((CH, D), jnp.float32),
        pltpu.VMEM((CH, D), jnp.float32),
        pltpu.SemaphoreType.DMA,
        pltpu.SemaphoreType.DMA,
        pltpu.SemaphoreType.DMA,
        pltpu.SemaphoreType.DMA,
    ],
)
def _sc_gather(src_hbm, pos_hbm, out_hbm, idx_v, b0, b1, g0, g1, o0, o1):
    wid = lax.axis_index("s") * NC + lax.axis_index("c")
    base = wid * RPW
    pltpu.sync_copy(pos_hbm.at[pl.ds(wid * NCH, NCH)], idx_v)
    bufs, gsem, osem = (b0, b1), (g0, g1), (o0, o1)
    gats = [None] * NCH
    outs = [None] * NCH
    gats[0] = pltpu.async_copy(src_hbm.at[idx_v.at[0]], bufs[0], gsem[0])
    for c in range(NCH):
        k = c % 2
        if c + 1 < NCH:
            nk = (c + 1) % 2
            if c >= 1:
                outs[c - 1].wait()
            gats[c + 1] = pltpu.async_copy(src_hbm.at[idx_v.at[c + 1]],
                                           bufs[nk], gsem[nk])
        gats[c].wait()
        outs[c] = pltpu.async_copy(bufs[k],
                                   out_hbm.at[pl.ds(base + c * CH, CH)],
                                   osem[k])
    outs[NCH - 2].wait()
    outs[NCH - 1].wait()


def _combine_body(rec_ref, maxp_ref, tok_ref, pb_ref, out_ref):
    out_ref[...] = (maxp_ref[...] * rec_ref[...] + tok_ref[...]
                    + pb_ref[...])


_combine_call = pl.pallas_call(
    _combine_body,
    grid=(B // T,),
    in_specs=[
        pl.BlockSpec((T, D), lambda i: (i, 0)),
        pl.BlockSpec((T, 1), lambda i: (i, 0)),
        pl.BlockSpec((T, D), lambda i: (i, 0)),
        pl.BlockSpec((1, D), lambda i: (0, 0)),
    ],
    out_specs=pl.BlockSpec((T, D), lambda i: (i, 0)),
    out_shape=jax.ShapeDtypeStruct((B, D), jnp.float32),
)


def kernel(activations, token_act, pre_b, enc, dec, router_b, router):
    pb2 = pre_b.reshape(1, D)
    maxp, eidx, wmean, pos, te, tv, prop = _router_call(
        activations, router, router_b.reshape(1, D))
    pos2 = pos.reshape(B // CH, CH)
    sorted_a = _sc_scatter(activations, pos2)
    lat_s, rec_s, wa = _gmm_call(
        te.reshape(NT), tv.reshape(NT), sorted_a, enc, pb2)
    rec_g = _sc_gather(rec_s, pos2)
    reconstruction = _combine_call(rec_g, maxp, token_act, pb2)
    full_latent = _sc_gather(lat_s, pos2)
    return (reconstruction, full_latent, wa.astype(bool),
            eidx.reshape(B), prop.reshape(N_EXP), wmean.reshape(N_EXP))


# speedup vs baseline: 1.2432x; 1.0857x over previous
import functools

import jax
import jax.numpy as jnp
from jax import lax
from jax.experimental import pallas as pl
from jax.experimental.pallas import tpu as pltpu
from jax.experimental.pallas import tpu_sc as plsc

N_EXP = 16
D = 1024
B = 8192
T = 256
NT = B // T + N_EXP
P = NT * T
NEG = -3.0e38

NC = 2
NS = 16
NW = NC * NS
RPW = B // NW
CH = 32
NCH = RPW // CH


H = D // 2


def _pack_bf16(x):
    b = lax.bitcast_convert_type(x, jnp.int32)
    r = b + 0x7FFF + (lax.shift_right_logical(b, 16) & 1)
    lo = r[:, :H]
    hi = r[:, H:]
    return (lax.shift_right_logical(lo, 16) & 0xFFFF) | (hi & (-65536))


def _unpack_bf16(p):
    flo = lax.bitcast_convert_type(lax.shift_left(p, 16), jnp.float32)
    fhi = lax.bitcast_convert_type(p & (-65536), jnp.float32)
    return jnp.concatenate([flo, fhi], axis=1)


def _router_body(act_ref, r_ref, rb_ref,
                 maxp_ref, idx_ref, wsum_ref, pos_ref, te_ref, tv_ref,
                 prop_ref, actb_ref, idx_sc, rank_sc, cnt_sc):
    i = pl.program_id(0)
    actb_ref[...] = _pack_bf16(act_ref[...])
    x = act_ref[...] - rb_ref[...]
    logits = jnp.dot(x, r_ref[...], preferred_element_type=jnp.float32)
    m = jnp.max(logits, axis=-1, keepdims=True)
    p = jnp.exp(logits - m)
    probs = p / jnp.sum(p, axis=-1, keepdims=True)
    maxp_ref[...] = jnp.max(probs, axis=-1, keepdims=True)
    lane = lax.broadcasted_iota(jnp.int32, (T, N_EXP), 1)
    eidx = jnp.min(jnp.where(logits == m, lane, N_EXP), axis=-1,
                   keepdims=True)
    idx_ref[...] = eidx
    idx_sc[pl.ds(i * T, T), :] = eidx
    onehot = (lane == eidx).astype(jnp.float32)

    @pl.when(i == 0)
    def _():
        cnt_sc[...] = jnp.zeros((1, N_EXP), jnp.float32)
        wsum_ref[...] = jnp.zeros((1, N_EXP), jnp.float32)

    tri = (lax.broadcasted_iota(jnp.int32, (T, T), 0)
           >= lax.broadcasted_iota(jnp.int32, (T, T), 1)).astype(jnp.float32)
    csum = jnp.dot(tri, onehot, preferred_element_type=jnp.float32) \
        + cnt_sc[...]
    rank = jnp.sum(onehot * csum, axis=-1, keepdims=True) - 1.0
    rank_sc[pl.ds(i * T, T), :] = rank.astype(jnp.int32)
    cnt_sc[...] = cnt_sc[...] + jnp.sum(onehot, axis=0, keepdims=True)
    wsum_ref[...] = wsum_ref[...] + jnp.sum(probs, axis=0, keepdims=True)

    @pl.when(i == B // T - 1)
    def _():
        wsum_ref[...] = wsum_ref[...] / float(B)
        prop_ref[...] = cnt_sc[...] / float(B)
        t_iota = lax.broadcasted_iota(jnp.int32, (1, NT), 1) \
            .astype(jnp.float32)
        te_f = jnp.zeros((1, NT), jnp.float32)
        start = 0.0
        starts = []
        bases = []
        cs = []
        for e in range(N_EXP):
            c = cnt_sc[0, e]
            cs.append(c)
            starts.append(start)
            bases.append(start / float(T))
            aligned = jnp.ceil(c / float(T)) * float(T)
            start = start + aligned
            te_f = te_f + (t_iota >= start / float(T)).astype(jnp.float32)
        te_f = jnp.minimum(te_f, float(N_EXP - 1))
        tv_f = jnp.zeros((1, NT), jnp.float32)
        for e in range(N_EXP):
            rem = cs[e] - (t_iota - bases[e]) * float(T)
            rem = jnp.clip(rem, 0.0, float(T))
            tv_f = tv_f + jnp.where(te_f == float(e), rem, 0.0)
        te_ref[...] = te_f.astype(jnp.int32)
        tv_ref[...] = tv_f.astype(jnp.int32)

        allidx = idx_sc[...]
        blane = lax.broadcasted_iota(jnp.int32, (B, N_EXP), 1)
        bonehot = (allidx == blane).astype(jnp.float32)
        erow = lax.broadcasted_iota(jnp.int32, (N_EXP, 1), 0)
        start_col = jnp.zeros((N_EXP, 1), jnp.float32)
        for e in range(N_EXP):
            start_col = start_col + jnp.where(erow == e, starts[e], 0.0)
        start_sel = jnp.dot(bonehot, start_col,
                            preferred_element_type=jnp.float32)
        pos_ref[...] = rank_sc[...] + start_sel.astype(jnp.int32)


_router_call = pl.pallas_call(
    _router_body,
    grid=(B // T,),
    in_specs=[
        pl.BlockSpec((T, D), lambda i: (i, 0)),
        pl.BlockSpec((D, N_EXP), lambda i: (0, 0)),
        pl.BlockSpec((1, D), lambda i: (0, 0)),
    ],
    out_specs=[
        pl.BlockSpec((T, 1), lambda i: (i, 0)),
        pl.BlockSpec((T, 1), lambda i: (i, 0)),
        pl.BlockSpec((1, N_EXP), lambda i: (0, 0)),
        pl.BlockSpec((B, 1), lambda i: (0, 0)),
        pl.BlockSpec((1, NT), lambda i: (0, 0)),
        pl.BlockSpec((1, NT), lambda i: (0, 0)),
        pl.BlockSpec((1, N_EXP), lambda i: (0, 0)),
        pl.BlockSpec((T, H), lambda i: (i, 0)),
    ],
    out_shape=[
        jax.ShapeDtypeStruct((B, 1), jnp.float32),
        jax.ShapeDtypeStruct((B, 1), jnp.int32),
        jax.ShapeDtypeStruct((1, N_EXP), jnp.float32),
        jax.ShapeDtypeStruct((B, 1), jnp.int32),
        jax.ShapeDtypeStruct((1, NT), jnp.int32),
        jax.ShapeDtypeStruct((1, NT), jnp.int32),
        jax.ShapeDtypeStruct((1, N_EXP), jnp.float32),
        jax.ShapeDtypeStruct((B, H), jnp.int32),
    ],
    scratch_shapes=[
        pltpu.VMEM((B, 1), jnp.int32),
        pltpu.VMEM((B, 1), jnp.int32),
        pltpu.VMEM((1, N_EXP), jnp.float32),
    ],
)


_sc_mesh = plsc.VectorSubcoreMesh(core_axis_name="c", subcore_axis_name="s")


@functools.partial(
    pl.kernel,
    mesh=_sc_mesh,
    out_type=jax.ShapeDtypeStruct((P, H), jnp.int32),
    scratch_types=[
        pltpu.VMEM((NCH, CH), jnp.int32),
        pltpu.VMEM((CH, H), jnp.int32),
        pltpu.VMEM((CH, H), jnp.int32),
        pltpu.SemaphoreType.DMA,
        pltpu.SemaphoreType.DMA,
        pltpu.SemaphoreType.DMA,
        pltpu.SemaphoreType.DMA,
    ],
)
def _sc_scatter(act_hbm, pos_hbm, out_hbm, idx_v, b0, b1, l0, l1, s0, s1):
    wid = lax.axis_index("s") * NC + lax.axis_index("c")
    base = wid * RPW
    pltpu.sync_copy(pos_hbm.at[pl.ds(wid * NCH, NCH)], idx_v)
    bufs, lsem, ssem = (b0, b1), (l0, l1), (s0, s1)
    loads = [None] * NCH
    scats = [None] * NCH
    loads[0] = pltpu.async_copy(act_hbm.at[pl.ds(base, CH)], bufs[0],
                                lsem[0])
    for c in range(NCH):
        k = c % 2
        if c + 1 < NCH:
            nk = (c + 1) % 2
            if c >= 1:
                scats[c - 1].wait()
            loads[c + 1] = pltpu.async_copy(
                act_hbm.at[pl.ds(base + (c + 1) * CH, CH)], bufs[nk],
                lsem[nk])
        loads[c].wait()
        scats[c] = pltpu.async_copy(bufs[k], out_hbm.at[idx_v.at[c]],
                                    ssem[k])
    scats[NCH - 2].wait()
    scats[NCH - 1].wait()


def _gmm_body(te_ref, tv_ref, xs_ref, enc_ref, pb_ref,
              lat_ref, rec_ref, wa_ref):
    t = pl.program_id(0)
    e = te_ref[t]
    tv = tv_ref[t]

    @pl.when(t == 0)
    def _():
        wa_ref[...] = jnp.full((N_EXP, D), NEG, jnp.float32)

    @pl.when(tv > 0)
    def _():
        x = _unpack_bf16(xs_ref[...]) - pb_ref[...]
        lat = jnp.maximum(
            jnp.dot(x, enc_ref[0], preferred_element_type=jnp.float32), 0.0)
        rec = lax.dot_general(
            lat, enc_ref[0], (((1,), (1,)), ((), ())),
            preferred_element_type=jnp.float32)
        lat_ref[...] = lat
        rec_ref[...] = _pack_bf16(rec)
        rows = lax.broadcasted_iota(jnp.int32, (T, 1), 0)
        masked = jnp.where(rows < tv, lat, NEG)
        m = jnp.max(masked, axis=0, keepdims=True)
        cur = wa_ref[pl.ds(e, 1), :]
        wa_ref[pl.ds(e, 1), :] = jnp.maximum(cur, m)

    @pl.when(t == NT - 1)
    def _():
        wa_ref[...] = jnp.where(wa_ref[...] > 0.001, 1.0, 0.0)


_gmm_call = pl.pallas_call(
    _gmm_body,
    grid_spec=pltpu.PrefetchScalarGridSpec(
        num_scalar_prefetch=2,
        grid=(NT,),
        in_specs=[
            pl.BlockSpec((T, H), lambda t, te, tv: (t, 0)),
            pl.BlockSpec((1, D, D), lambda t, te, tv: (te[t], 0, 0)),
            pl.BlockSpec((1, D), lambda t, te, tv: (0, 0)),
        ],
        out_specs=[
            pl.BlockSpec((T, D), lambda t, te, tv: (t, 0)),
            pl.BlockSpec((T, H), lambda t, te, tv: (t, 0)),
            pl.BlockSpec((N_EXP, D), lambda t, te, tv: (0, 0)),
        ],
    ),
    out_shape=[
        jax.ShapeDtypeStruct((P, D), jnp.float32),
        jax.ShapeDtypeStruct((P, H), jnp.int32),
        jax.ShapeDtypeStruct((N_EXP, D), jnp.float32),
    ],
)


def _make_sc_gather(dtype, width):
  @functools.partial(
      pl.kernel,
      mesh=_sc_mesh,
      out_type=jax.ShapeDtypeStruct((B, width), dtype),
      scratch_types=[
          pltpu.VMEM((NCH, CH), jnp.int32),
          pltpu.VMEM((CH, width), dtype),
          pltpu.VMEM((CH, width), dtype),
          pltpu.SemaphoreType.DMA,
          pltpu.SemaphoreType.DMA,
          pltpu.SemaphoreType.DMA,
          pltpu.SemaphoreType.DMA,
      ],
  )
  def _sc_gather(src_hbm, pos_hbm, out_hbm, idx_v, b0, b1, g0, g1, o0, o1):
      wid = lax.axis_index("s") * NC + lax.axis_index("c")
      base = wid * RPW
      pltpu.sync_copy(pos_hbm.at[pl.ds(wid * NCH, NCH)], idx_v)
      bufs, gsem, osem = (b0, b1), (g0, g1), (o0, o1)
      gats = [None] * NCH
      outs = [None] * NCH
      gats[0] = pltpu.async_copy(src_hbm.at[idx_v.at[0]], bufs[0], gsem[0])
      for c in range(NCH):
          k = c % 2
          if c + 1 < NCH:
              nk = (c + 1) % 2
              if c >= 1:
                  outs[c - 1].wait()
              gats[c + 1] = pltpu.async_copy(src_hbm.at[idx_v.at[c + 1]],
                                             bufs[nk], gsem[nk])
          gats[c].wait()
          outs[c] = pltpu.async_copy(bufs[k],
                                     out_hbm.at[pl.ds(base + c * CH, CH)],
                                     osem[k])
      outs[NCH - 2].wait()
      outs[NCH - 1].wait()

  return _sc_gather


_sc_gather_f32 = _make_sc_gather(jnp.float32, D)
_sc_gather_i32 = _make_sc_gather(jnp.int32, H)


def _combine_body(rec_ref, maxp_ref, tok_ref, pb_ref, out_ref):
    out_ref[...] = (maxp_ref[...] * _unpack_bf16(rec_ref[...])
                    + tok_ref[...] + pb_ref[...])


_combine_call = pl.pallas_call(
    _combine_body,
    grid=(B // T,),
    in_specs=[
        pl.BlockSpec((T, H), lambda i: (i, 0)),
        pl.BlockSpec((T, 1), lambda i: (i, 0)),
        pl.BlockSpec((T, D), lambda i: (i, 0)),
        pl.BlockSpec((1, D), lambda i: (0, 0)),
    ],
    out_specs=pl.BlockSpec((T, D), lambda i: (i, 0)),
    out_shape=jax.ShapeDtypeStruct((B, D), jnp.float32),
)


def kernel(activations, token_act, pre_b, enc, dec, router_b, router):
    pb2 = pre_b.reshape(1, D)
    maxp, eidx, wmean, pos, te, tv, prop, actb = _router_call(
        activations, router, router_b.reshape(1, D))
    pos2 = pos.reshape(B // CH, CH)
    sorted_a = _sc_scatter(actb, pos2)
    lat_s, rec_s, wa = _gmm_call(
        te.reshape(NT), tv.reshape(NT), sorted_a, enc, pb2)
    rec_g = _sc_gather_i32(rec_s, pos2)
    reconstruction = _combine_call(rec_g, maxp, token_act, pb2)
    full_latent = _sc_gather_f32(lat_s, pos2)
    return (reconstruction, full_latent, wa.astype(bool),
            eidx.reshape(B), prop.reshape(N_EXP), wmean.reshape(N_EXP))


# speedup vs baseline: 1.2553x; 1.0097x over previous
import functools

import jax
import jax.numpy as jnp
from jax import lax
from jax.experimental import pallas as pl
from jax.experimental.pallas import tpu as pltpu
from jax.experimental.pallas import tpu_sc as plsc

N_EXP = 16
D = 1024
B = 8192
T = 256
NT = B // T + N_EXP
P = NT * T
NEG = -3.0e38

NC = 2
NS = 16
NW = NC * NS
RPW = B // NW
CH = 32
NCH = RPW // CH


H = D // 2


def _pack_bf16(x):
    b = lax.bitcast_convert_type(x, jnp.int32)
    r = b + 0x7FFF + (lax.shift_right_logical(b, 16) & 1)
    lo = r[:, :H]
    hi = r[:, H:]
    return (lax.shift_right_logical(lo, 16) & 0xFFFF) | (hi & (-65536))


def _unpack_bf16(p):
    flo = lax.bitcast_convert_type(lax.shift_left(p, 16), jnp.float32)
    fhi = lax.bitcast_convert_type(p & (-65536), jnp.float32)
    return jnp.concatenate([flo, fhi], axis=1)


def _router_body(act_ref, r_ref, rb_ref,
                 maxp_ref, idx_ref, wsum_ref, pos_ref, te_ref, tv_ref,
                 prop_ref, actb_ref, tm_ref, idx_sc, rank_sc, cnt_sc):
    i = pl.program_id(0)
    actb_ref[...] = _pack_bf16(act_ref[...])
    x = act_ref[...] - rb_ref[...]
    logits = jnp.dot(x, r_ref[...], preferred_element_type=jnp.float32)
    m = jnp.max(logits, axis=-1, keepdims=True)
    p = jnp.exp(logits - m)
    probs = p / jnp.sum(p, axis=-1, keepdims=True)
    maxp_ref[...] = jnp.max(probs, axis=-1, keepdims=True)
    lane = lax.broadcasted_iota(jnp.int32, (T, N_EXP), 1)
    eidx = jnp.min(jnp.where(logits == m, lane, N_EXP), axis=-1,
                   keepdims=True)
    idx_ref[...] = eidx
    idx_sc[pl.ds(i * T, T), :] = eidx
    onehot = (lane == eidx).astype(jnp.float32)

    @pl.when(i == 0)
    def _():
        cnt_sc[...] = jnp.zeros((1, N_EXP), jnp.float32)
        wsum_ref[...] = jnp.zeros((1, N_EXP), jnp.float32)

    tri = (lax.broadcasted_iota(jnp.int32, (T, T), 0)
           >= lax.broadcasted_iota(jnp.int32, (T, T), 1)).astype(jnp.float32)
    csum = jnp.dot(tri.astype(jnp.bfloat16), onehot.astype(jnp.bfloat16),
                   preferred_element_type=jnp.float32) + cnt_sc[...]
    rank = jnp.sum(onehot * csum, axis=-1, keepdims=True) - 1.0
    rank_sc[pl.ds(i * T, T), :] = rank.astype(jnp.int32)
    cnt_sc[...] = cnt_sc[...] + jnp.sum(onehot, axis=0, keepdims=True)
    wsum_ref[...] = wsum_ref[...] + jnp.sum(probs, axis=0, keepdims=True)

    @pl.when(i == B // T - 1)
    def _():
        wsum_ref[...] = wsum_ref[...] / float(B)
        prop_ref[...] = cnt_sc[...] / float(B)
        t_iota = lax.broadcasted_iota(jnp.int32, (1, NT), 1) \
            .astype(jnp.float32)
        te_f = jnp.zeros((1, NT), jnp.float32)
        start = 0.0
        starts = []
        bases = []
        cs = []
        for e in range(N_EXP):
            c = cnt_sc[0, e]
            cs.append(c)
            starts.append(start)
            bases.append(start / float(T))
            aligned = jnp.ceil(c / float(T)) * float(T)
            start = start + aligned
            te_f = te_f + (t_iota >= start / float(T)).astype(jnp.float32)
        te_f = jnp.minimum(te_f, float(N_EXP - 1))
        used_m1 = start / float(T) - 1.0
        te_last = 0.0
        cum = 0.0
        for e in range(N_EXP):
            cum = cum + jnp.ceil(cs[e] / float(T))
            te_last = te_last + jnp.where(used_m1 >= cum, 1.0, 0.0)
        te_f = jnp.minimum(te_f, te_last)
        tv_f = jnp.zeros((1, NT), jnp.float32)
        for e in range(N_EXP):
            rem = cs[e] - (t_iota - bases[e]) * float(T)
            rem = jnp.clip(rem, 0.0, float(T))
            tv_f = tv_f + jnp.where(te_f == float(e), rem, 0.0)
        te_ref[...] = te_f.astype(jnp.int32)
        tv_ref[...] = tv_f.astype(jnp.int32)
        used = start / float(T)
        tm_ref[...] = jnp.minimum(t_iota, used - 1.0).astype(jnp.int32)

        allidx = idx_sc[...]
        blane = lax.broadcasted_iota(jnp.int32, (B, N_EXP), 1)
        bonehot = (allidx == blane).astype(jnp.float32)
        erow = lax.broadcasted_iota(jnp.int32, (N_EXP, 1), 0)
        start_col = jnp.zeros((N_EXP, 1), jnp.float32)
        for e in range(N_EXP):
            start_col = start_col + jnp.where(erow == e, starts[e], 0.0)
        start_sel = jnp.dot(bonehot, start_col,
                            preferred_element_type=jnp.float32)
        pos_ref[...] = rank_sc[...] + start_sel.astype(jnp.int32)


_router_call = pl.pallas_call(
    _router_body,
    grid=(B // T,),
    in_specs=[
        pl.BlockSpec((T, D), lambda i: (i, 0)),
        pl.BlockSpec((D, N_EXP), lambda i: (0, 0)),
        pl.BlockSpec((1, D), lambda i: (0, 0)),
    ],
    out_specs=[
        pl.BlockSpec((T, 1), lambda i: (i, 0)),
        pl.BlockSpec((T, 1), lambda i: (i, 0)),
        pl.BlockSpec((1, N_EXP), lambda i: (0, 0)),
        pl.BlockSpec((B, 1), lambda i: (0, 0)),
        pl.BlockSpec((1, NT), lambda i: (0, 0)),
        pl.BlockSpec((1, NT), lambda i: (0, 0)),
        pl.BlockSpec((1, N_EXP), lambda i: (0, 0)),
        pl.BlockSpec((T, H), lambda i: (i, 0)),
        pl.BlockSpec((1, NT), lambda i: (0, 0)),
    ],
    out_shape=[
        jax.ShapeDtypeStruct((B, 1), jnp.float32),
        jax.ShapeDtypeStruct((B, 1), jnp.int32),
        jax.ShapeDtypeStruct((1, N_EXP), jnp.float32),
        jax.ShapeDtypeStruct((B, 1), jnp.int32),
        jax.ShapeDtypeStruct((1, NT), jnp.int32),
        jax.ShapeDtypeStruct((1, NT), jnp.int32),
        jax.ShapeDtypeStruct((1, N_EXP), jnp.float32),
        jax.ShapeDtypeStruct((B, H), jnp.int32),
        jax.ShapeDtypeStruct((1, NT), jnp.int32),
    ],
    scratch_shapes=[
        pltpu.VMEM((B, 1), jnp.int32),
        pltpu.VMEM((B, 1), jnp.int32),
        pltpu.VMEM((1, N_EXP), jnp.float32),
    ],
)


_sc_mesh = plsc.VectorSubcoreMesh(core_axis_name="c", subcore_axis_name="s")


@functools.partial(
    pl.kernel,
    mesh=_sc_mesh,
    out_type=jax.ShapeDtypeStruct((P, H), jnp.int32),
    scratch_types=[
        pltpu.VMEM((NCH, CH), jnp.int32),
        pltpu.VMEM((CH, H), jnp.int32),
        pltpu.VMEM((CH, H), jnp.int32),
        pltpu.SemaphoreType.DMA,
        pltpu.SemaphoreType.DMA,
        pltpu.SemaphoreType.DMA,
        pltpu.SemaphoreType.DMA,
    ],
)
def _sc_scatter(act_hbm, pos_hbm, out_hbm, idx_v, b0, b1, l0, l1, s0, s1):
    wid = lax.axis_index("s") * NC + lax.axis_index("c")
    base = wid * RPW
    pltpu.sync_copy(pos_hbm.at[pl.ds(wid * NCH, NCH)], idx_v)
    bufs, lsem, ssem = (b0, b1), (l0, l1), (s0, s1)
    loads = [None] * NCH
    scats = [None] * NCH
    loads[0] = pltpu.async_copy(act_hbm.at[pl.ds(base, CH)], bufs[0],
                                lsem[0])
    for c in range(NCH):
        k = c % 2
        if c + 1 < NCH:
            nk = (c + 1) % 2
            if c >= 1:
                scats[c - 1].wait()
            loads[c + 1] = pltpu.async_copy(
                act_hbm.at[pl.ds(base + (c + 1) * CH, CH)], bufs[nk],
                lsem[nk])
        loads[c].wait()
        scats[c] = pltpu.async_copy(bufs[k], out_hbm.at[idx_v.at[c]],
                                    ssem[k])
    scats[NCH - 2].wait()
    scats[NCH - 1].wait()


def _gmm_body(te_ref, tv_ref, tm_ref, xs_ref, enc_ref, pb_ref,
              lat_ref, rec_ref, wa_ref):
    t = pl.program_id(0)
    e = te_ref[t]
    tv = tv_ref[t]

    @pl.when(t == 0)
    def _():
        wa_ref[...] = jnp.full((N_EXP, D), NEG, jnp.float32)

    @pl.when(tv > 0)
    def _():
        x = _unpack_bf16(xs_ref[...]) - pb_ref[...]
        lat = jnp.maximum(
            jnp.dot(x, enc_ref[0], preferred_element_type=jnp.float32), 0.0)
        rec = lax.dot_general(
            lat, enc_ref[0], (((1,), (1,)), ((), ())),
            preferred_element_type=jnp.float32)
        lat_ref[...] = lat
        rec_ref[...] = _pack_bf16(rec)
        rows = lax.broadcasted_iota(jnp.int32, (T, 1), 0)
        masked = jnp.where(rows < tv, lat, NEG)
        m = jnp.max(masked, axis=0, keepdims=True)
        cur = wa_ref[pl.ds(e, 1), :]
        wa_ref[pl.ds(e, 1), :] = jnp.maximum(cur, m)

    @pl.when(t == NT - 1)
    def _():
        wa_ref[...] = jnp.where(wa_ref[...] > 0.001, 1.0, 0.0)


_gmm_call = pl.pallas_call(
    _gmm_body,
    grid_spec=pltpu.PrefetchScalarGridSpec(
        num_scalar_prefetch=3,
        grid=(NT,),
        in_specs=[
            pl.BlockSpec((T, H), lambda t, te, tv, tm: (tm[t], 0)),
            pl.BlockSpec((1, D, D), lambda t, te, tv, tm: (te[t], 0, 0)),
            pl.BlockSpec((1, D), lambda t, te, tv, tm: (0, 0)),
        ],
        out_specs=[
            pl.BlockSpec((T, D), lambda t, te, tv, tm: (t, 0)),
            pl.BlockSpec((T, H), lambda t, te, tv, tm: (t, 0)),
            pl.BlockSpec((N_EXP, D), lambda t, te, tv, tm: (0, 0)),
        ],
    ),
    out_shape=[
        jax.ShapeDtypeStruct((P, D), jnp.float32),
        jax.ShapeDtypeStruct((P, H), jnp.int32),
        jax.ShapeDtypeStruct((N_EXP, D), jnp.float32),
    ],
)


def _make_sc_gather(dtype, width):
  @functools.partial(
      pl.kernel,
      mesh=_sc_mesh,
      out_type=jax.ShapeDtypeStruct((B, width), dtype),
      scratch_types=[
          pltpu.VMEM((NCH, CH), jnp.int32),
          pltpu.VMEM((CH, width), dtype),
          pltpu.VMEM((CH, width), dtype),
          pltpu.SemaphoreType.DMA,
          pltpu.SemaphoreType.DMA,
          pltpu.SemaphoreType.DMA,
          pltpu.SemaphoreType.DMA,
      ],
  )
  def _sc_gather(src_hbm, pos_hbm, out_hbm, idx_v, b0, b1, g0, g1, o0, o1):
      wid = lax.axis_index("s") * NC + lax.axis_index("c")
      base = wid * RPW
      pltpu.sync_copy(pos_hbm.at[pl.ds(wid * NCH, NCH)], idx_v)
      bufs, gsem, osem = (b0, b1), (g0, g1), (o0, o1)
      gats = [None] * NCH
      outs = [None] * NCH
      gats[0] = pltpu.async_copy(src_hbm.at[idx_v.at[0]], bufs[0], gsem[0])
      for c in range(NCH):
          k = c % 2
          if c + 1 < NCH:
              nk = (c + 1) % 2
              if c >= 1:
                  outs[c - 1].wait()
              gats[c + 1] = pltpu.async_copy(src_hbm.at[idx_v.at[c + 1]],
                                             bufs[nk], gsem[nk])
          gats[c].wait()
          outs[c] = pltpu.async_copy(bufs[k],
                                     out_hbm.at[pl.ds(base + c * CH, CH)],
                                     osem[k])
      outs[NCH - 2].wait()
      outs[NCH - 1].wait()

  return _sc_gather


_sc_gather_f32 = _make_sc_gather(jnp.float32, D)
_sc_gather_i32 = _make_sc_gather(jnp.int32, H)


def _combine_body(rec_ref, maxp_ref, tok_ref, pb_ref, out_ref):
    out_ref[...] = (maxp_ref[...] * _unpack_bf16(rec_ref[...])
                    + tok_ref[...] + pb_ref[...])


_combine_call = pl.pallas_call(
    _combine_body,
    grid=(B // T,),
    in_specs=[
        pl.BlockSpec((T, H), lambda i: (i, 0)),
        pl.BlockSpec((T, 1), lambda i: (i, 0)),
        pl.BlockSpec((T, D), lambda i: (i, 0)),
        pl.BlockSpec((1, D), lambda i: (0, 0)),
    ],
    out_specs=pl.BlockSpec((T, D), lambda i: (i, 0)),
    out_shape=jax.ShapeDtypeStruct((B, D), jnp.float32),
)


def kernel(activations, token_act, pre_b, enc, dec, router_b, router):
    pb2 = pre_b.reshape(1, D)
    maxp, eidx, wmean, pos, te, tv, prop, actb, tm = _router_call(
        activations, router, router_b.reshape(1, D))
    pos2 = pos.reshape(B // CH, CH)
    sorted_a = _sc_scatter(actb, pos2)
    lat_s, rec_s, wa = _gmm_call(
        te.reshape(NT), tv.reshape(NT), tm.reshape(NT), sorted_a, enc, pb2)
    rec_g = _sc_gather_i32(rec_s, pos2)
    reconstruction = _combine_call(rec_g, maxp, token_act, pb2)
    full_latent = _sc_gather_f32(lat_s, pos2)
    return (reconstruction, full_latent, wa.astype(bool),
            eidx.reshape(B), prop.reshape(N_EXP), wmean.reshape(N_EXP))


# speedup vs baseline: 1.2875x; 1.0257x over previous
import functools

import jax
import jax.numpy as jnp
from jax import lax
from jax.experimental import pallas as pl
from jax.experimental.pallas import tpu as pltpu
from jax.experimental.pallas import tpu_sc as plsc

N_EXP = 16
D = 1024
B = 8192
T = 256
NT = B // T + N_EXP
P = NT * T
NEG = -3.0e38

NC = 2
NS = 16
NW = NC * NS
RPW = B // NW
CH = 32
NCH = RPW // CH


H = D // 2


def _pack_bf16(x):
    b = lax.bitcast_convert_type(x, jnp.int32)
    r = b + 0x7FFF + (lax.shift_right_logical(b, 16) & 1)
    lo = r[:, :H]
    hi = r[:, H:]
    return lax.shift_right_logical(lo, 16) | (hi & (-65536))


def _pack_bf16_trunc(x):
    b = lax.bitcast_convert_type(x, jnp.int32)
    return lax.shift_right_logical(b[:, :H], 16) | (b[:, H:] & (-65536))


def _unpack_bf16(p):
    flo = lax.bitcast_convert_type(lax.shift_left(p, 16), jnp.float32)
    fhi = lax.bitcast_convert_type(p & (-65536), jnp.float32)
    return jnp.concatenate([flo, fhi], axis=1)


def _router_body(act_ref, r_ref, rb_ref,
                 maxp_ref, idx_ref, wsum_ref, pos_ref, te_ref, tv_ref,
                 prop_ref, actb_ref, tm_ref, idx_sc, rank_sc, cnt_sc):
    i = pl.program_id(0)
    actb_ref[...] = _pack_bf16_trunc(act_ref[...])
    x = act_ref[...] - rb_ref[...]
    logits = jnp.dot(x, r_ref[...], preferred_element_type=jnp.float32)
    m = jnp.max(logits, axis=-1, keepdims=True)
    p = jnp.exp(logits - m)
    probs = p / jnp.sum(p, axis=-1, keepdims=True)
    maxp_ref[...] = jnp.max(probs, axis=-1, keepdims=True)
    lane = lax.broadcasted_iota(jnp.int32, (T, N_EXP), 1)
    eidx = jnp.min(jnp.where(logits == m, lane, N_EXP), axis=-1,
                   keepdims=True)
    idx_ref[...] = eidx
    idx_sc[pl.ds(i * T, T), :] = eidx
    onehot = (lane == eidx).astype(jnp.float32)

    @pl.when(i == 0)
    def _():
        cnt_sc[...] = jnp.zeros((1, N_EXP), jnp.float32)
        wsum_ref[...] = jnp.zeros((1, N_EXP), jnp.float32)

    tri = (lax.broadcasted_iota(jnp.int32, (T, T), 0)
           >= lax.broadcasted_iota(jnp.int32, (T, T), 1)).astype(jnp.float32)
    csum = jnp.dot(tri.astype(jnp.bfloat16), onehot.astype(jnp.bfloat16),
                   preferred_element_type=jnp.float32) + cnt_sc[...]
    rank = jnp.sum(onehot * csum, axis=-1, keepdims=True) - 1.0
    rank_sc[pl.ds(i * T, T), :] = rank.astype(jnp.int32)
    cnt_sc[...] = cnt_sc[...] + jnp.sum(onehot, axis=0, keepdims=True)
    wsum_ref[...] = wsum_ref[...] + jnp.sum(probs, axis=0, keepdims=True)

    @pl.when(i == B // T - 1)
    def _():
        wsum_ref[...] = wsum_ref[...] / float(B)
        prop_ref[...] = cnt_sc[...] / float(B)
        t_iota = lax.broadcasted_iota(jnp.int32, (1, NT), 1) \
            .astype(jnp.float32)
        te_f = jnp.zeros((1, NT), jnp.float32)
        start = 0.0
        starts = []
        bases = []
        cs = []
        for e in range(N_EXP):
            c = cnt_sc[0, e]
            cs.append(c)
            starts.append(start)
            bases.append(start / float(T))
            aligned = jnp.ceil(c / float(T)) * float(T)
            start = start + aligned
            te_f = te_f + (t_iota >= start / float(T)).astype(jnp.float32)
        te_f = jnp.minimum(te_f, float(N_EXP - 1))
        used_m1 = start / float(T) - 1.0
        te_last = 0.0
        cum = 0.0
        for e in range(N_EXP):
            cum = cum + jnp.ceil(cs[e] / float(T))
            te_last = te_last + jnp.where(used_m1 >= cum, 1.0, 0.0)
        te_f = jnp.minimum(te_f, te_last)
        tv_f = jnp.zeros((1, NT), jnp.float32)
        for e in range(N_EXP):
            rem = cs[e] - (t_iota - bases[e]) * float(T)
            rem = jnp.clip(rem, 0.0, float(T))
            tv_f = tv_f + jnp.where(te_f == float(e), rem, 0.0)
        te_ref[...] = te_f.astype(jnp.int32)
        tv_ref[...] = tv_f.astype(jnp.int32)
        used = start / float(T)
        tm_ref[...] = jnp.minimum(t_iota, used - 1.0).astype(jnp.int32)

        allidx = idx_sc[...]
        blane = lax.broadcasted_iota(jnp.int32, (B, N_EXP), 1)
        bonehot = (allidx == blane).astype(jnp.float32)
        erow = lax.broadcasted_iota(jnp.int32, (N_EXP, 1), 0)
        start_col = jnp.zeros((N_EXP, 1), jnp.float32)
        for e in range(N_EXP):
            start_col = start_col + jnp.where(erow == e, starts[e], 0.0)
        start_sel = jnp.dot(bonehot, start_col,
                            preferred_element_type=jnp.float32)
        pos_ref[...] = rank_sc[...] + start_sel.astype(jnp.int32)


_router_call = pl.pallas_call(
    _router_body,
    grid=(B // T,),
    in_specs=[
        pl.BlockSpec((T, D), lambda i: (i, 0)),
        pl.BlockSpec((D, N_EXP), lambda i: (0, 0)),
        pl.BlockSpec((1, D), lambda i: (0, 0)),
    ],
    out_specs=[
        pl.BlockSpec((T, 1), lambda i: (i, 0)),
        pl.BlockSpec((T, 1), lambda i: (i, 0)),
        pl.BlockSpec((1, N_EXP), lambda i: (0, 0)),
        pl.BlockSpec((B, 1), lambda i: (0, 0)),
        pl.BlockSpec((1, NT), lambda i: (0, 0)),
        pl.BlockSpec((1, NT), lambda i: (0, 0)),
        pl.BlockSpec((1, N_EXP), lambda i: (0, 0)),
        pl.BlockSpec((T, H), lambda i: (i, 0)),
        pl.BlockSpec((1, NT), lambda i: (0, 0)),
    ],
    out_shape=[
        jax.ShapeDtypeStruct((B, 1), jnp.float32),
        jax.ShapeDtypeStruct((B, 1), jnp.int32),
        jax.ShapeDtypeStruct((1, N_EXP), jnp.float32),
        jax.ShapeDtypeStruct((B, 1), jnp.int32),
        jax.ShapeDtypeStruct((1, NT), jnp.int32),
        jax.ShapeDtypeStruct((1, NT), jnp.int32),
        jax.ShapeDtypeStruct((1, N_EXP), jnp.float32),
        jax.ShapeDtypeStruct((B, H), jnp.int32),
        jax.ShapeDtypeStruct((1, NT), jnp.int32),
    ],
    scratch_shapes=[
        pltpu.VMEM((B, 1), jnp.int32),
        pltpu.VMEM((B, 1), jnp.int32),
        pltpu.VMEM((1, N_EXP), jnp.float32),
    ],
)


_sc_mesh = plsc.VectorSubcoreMesh(core_axis_name="c", subcore_axis_name="s")


@functools.partial(
    pl.kernel,
    mesh=_sc_mesh,
    out_type=jax.ShapeDtypeStruct((P, H), jnp.int32),
    scratch_types=[
        pltpu.VMEM((NCH, CH), jnp.int32),
        pltpu.VMEM((CH, H), jnp.int32),
        pltpu.VMEM((CH, H), jnp.int32),
        pltpu.SemaphoreType.DMA,
        pltpu.SemaphoreType.DMA,
        pltpu.SemaphoreType.DMA,
        pltpu.SemaphoreType.DMA,
    ],
)
def _sc_scatter(act_hbm, pos_hbm, out_hbm, idx_v, b0, b1, l0, l1, s0, s1):
    wid = lax.axis_index("s") * NC + lax.axis_index("c")
    base = wid * RPW
    pltpu.sync_copy(pos_hbm.at[pl.ds(wid * NCH, NCH)], idx_v)
    bufs, lsem, ssem = (b0, b1), (l0, l1), (s0, s1)
    loads = [None] * NCH
    scats = [None] * NCH
    loads[0] = pltpu.async_copy(act_hbm.at[pl.ds(base, CH)], bufs[0],
                                lsem[0])
    for c in range(NCH):
        k = c % 2
        if c + 1 < NCH:
            nk = (c + 1) % 2
            if c >= 1:
                scats[c - 1].wait()
            loads[c + 1] = pltpu.async_copy(
                act_hbm.at[pl.ds(base + (c + 1) * CH, CH)], bufs[nk],
                lsem[nk])
        loads[c].wait()
        scats[c] = pltpu.async_copy(bufs[k], out_hbm.at[idx_v.at[c]],
                                    ssem[k])
    scats[NCH - 2].wait()
    scats[NCH - 1].wait()


def _gmm_body(te_ref, tv_ref, tm_ref, xs_ref, enc_ref, pb_ref,
              lat_ref, rec_ref, wa_ref):
    t = pl.program_id(0)
    e = te_ref[t]
    tv = tv_ref[t]

    @pl.when(t == 0)
    def _():
        wa_ref[...] = jnp.full((N_EXP, D), NEG, jnp.float32)

    @pl.when(tv > 0)
    def _():
        x = _unpack_bf16(xs_ref[...]) - pb_ref[...]
        lat = jnp.maximum(
            jnp.dot(x, enc_ref[0], preferred_element_type=jnp.float32), 0.0)
        rec = lax.dot_general(
            lat, enc_ref[0], (((1,), (1,)), ((), ())),
            preferred_element_type=jnp.float32)
        lat_ref[...] = lat
        rec_ref[...] = _pack_bf16(rec)
        rows = lax.broadcasted_iota(jnp.int32, (T, 1), 0)
        masked = jnp.where(rows < tv, lat, NEG)
        m = jnp.max(masked, axis=0, keepdims=True)
        cur = wa_ref[pl.ds(e, 1), :]
        wa_ref[pl.ds(e, 1), :] = jnp.maximum(cur, m)

    @pl.when(t == NT - 1)
    def _():
        wa_ref[...] = jnp.where(wa_ref[...] > 0.001, 1.0, 0.0)


_gmm_call = pl.pallas_call(
    _gmm_body,
    grid_spec=pltpu.PrefetchScalarGridSpec(
        num_scalar_prefetch=3,
        grid=(NT,),
        in_specs=[
            pl.BlockSpec((T, H), lambda t, te, tv, tm: (tm[t], 0)),
            pl.BlockSpec((1, D, D), lambda t, te, tv, tm: (te[t], 0, 0)),
            pl.BlockSpec((1, D), lambda t, te, tv, tm: (0, 0)),
        ],
        out_specs=[
            pl.BlockSpec((T, D), lambda t, te, tv, tm: (t, 0)),
            pl.BlockSpec((T, H), lambda t, te, tv, tm: (t, 0)),
            pl.BlockSpec((N_EXP, D), lambda t, te, tv, tm: (0, 0)),
        ],
    ),
    out_shape=[
        jax.ShapeDtypeStruct((P, D), jnp.float32),
        jax.ShapeDtypeStruct((P, H), jnp.int32),
        jax.ShapeDtypeStruct((N_EXP, D), jnp.float32),
    ],
)


def _make_sc_gather(dtype, width):
  @functools.partial(
      pl.kernel,
      mesh=_sc_mesh,
      out_type=jax.ShapeDtypeStruct((B, width), dtype),
      scratch_types=[
          pltpu.VMEM((NCH, CH), jnp.int32),
          pltpu.VMEM((CH, width), dtype),
          pltpu.VMEM((CH, width), dtype),
          pltpu.SemaphoreType.DMA,
          pltpu.SemaphoreType.DMA,
          pltpu.SemaphoreType.DMA,
          pltpu.SemaphoreType.DMA,
      ],
  )
  def _sc_gather(src_hbm, pos_hbm, out_hbm, idx_v, b0, b1, g0, g1, o0, o1):
      wid = lax.axis_index("s") * NC + lax.axis_index("c")
      base = wid * RPW
      pltpu.sync_copy(pos_hbm.at[pl.ds(wid * NCH, NCH)], idx_v)
      bufs, gsem, osem = (b0, b1), (g0, g1), (o0, o1)
      gats = [None] * NCH
      outs = [None] * NCH
      gats[0] = pltpu.async_copy(src_hbm.at[idx_v.at[0]], bufs[0], gsem[0])
      for c in range(NCH):
          k = c % 2
          if c + 1 < NCH:
              nk = (c + 1) % 2
              if c >= 1:
                  outs[c - 1].wait()
              gats[c + 1] = pltpu.async_copy(src_hbm.at[idx_v.at[c + 1]],
                                             bufs[nk], gsem[nk])
          gats[c].wait()
          outs[c] = pltpu.async_copy(bufs[k],
                                     out_hbm.at[pl.ds(base + c * CH, CH)],
                                     osem[k])
      outs[NCH - 2].wait()
      outs[NCH - 1].wait()

  return _sc_gather


_sc_gather_f32 = _make_sc_gather(jnp.float32, D)
_sc_gather_i32 = _make_sc_gather(jnp.int32, H)


def _combine_body(rec_ref, maxp_ref, tok_ref, pb_ref, out_ref):
    out_ref[...] = (maxp_ref[...] * _unpack_bf16(rec_ref[...])
                    + tok_ref[...] + pb_ref[...])


_combine_call = pl.pallas_call(
    _combine_body,
    grid=(B // T,),
    in_specs=[
        pl.BlockSpec((T, H), lambda i: (i, 0)),
        pl.BlockSpec((T, 1), lambda i: (i, 0)),
        pl.BlockSpec((T, D), lambda i: (i, 0)),
        pl.BlockSpec((1, D), lambda i: (0, 0)),
    ],
    out_specs=pl.BlockSpec((T, D), lambda i: (i, 0)),
    out_shape=jax.ShapeDtypeStruct((B, D), jnp.float32),
)


def kernel(activations, token_act, pre_b, enc, dec, router_b, router):
    pb2 = pre_b.reshape(1, D)
    maxp, eidx, wmean, pos, te, tv, prop, actb, tm = _router_call(
        activations, router, router_b.reshape(1, D))
    pos2 = pos.reshape(B // CH, CH)
    sorted_a = _sc_scatter(actb, pos2)
    lat_s, rec_s, wa = _gmm_call(
        te.reshape(NT), tv.reshape(NT), tm.reshape(NT), sorted_a, enc, pb2)
    rec_g = _sc_gather_i32(rec_s, pos2)
    reconstruction = _combine_call(rec_g, maxp, token_act, pb2)
    full_latent = _sc_gather_f32(lat_s, pos2)
    return (reconstruction, full_latent, wa.astype(bool),
            eidx.reshape(B), prop.reshape(N_EXP), wmean.reshape(N_EXP))
